# baseline (speedup 1.0000x reference)
import functools

import jax
import jax.numpy as jnp
from jax import lax
from jax.experimental import pallas as pl
from jax.experimental.pallas import tpu as pltpu
from jax.experimental.pallas import tpu_sc as plsc

N = 50000
IN_DIM = 19
HIDDEN = 64

NC = 2
NS = 16
HALF = 25000
ACC_ROWS = 25088
TRASH = 25080
CHUNK = 96
SUP = 8
SUPE = SUP * CHUNK
T = 66
WB = 1560
ROWS_PT = ACC_ROWS // NS


NSLOT = 4


def _sc_agg_body(h_hbm, src_hbm, dst_hbm, out_hbm,
                 srcv, dstv, ld0, ld1, ld2, ld3, r0, r1, r2, r3,
                 acc_sh, gs0, gs1, gs2, gs3, ss0, ss1, ss2, ss3):
    c = lax.axis_index("c")
    s = lax.axis_index("s")
    base = c * HALF
    LD = [ld0, ld1, ld2, ld3]
    R = [r0, r1, r2, r3]
    GS = [gs0, gs1, gs2, gs3]
    SS = [ss0, ss1, ss2, ss3]

    def zrow(i, carry):
        for j in range(HIDDEN // 16):
            r0[i, pl.ds(j * 16, 16)] = jnp.zeros((16,), jnp.float32)
        return carry
    lax.fori_loop(0, CHUNK, zrow, 0)
    rbase = s * ROWS_PT

    def zsplat(i, carry):
        pltpu.sync_copy(r0, acc_sh.at[pl.ds(rbase + i * CHUNK, CHUNK)])
        return carry
    lax.fori_loop(0, 16, zsplat, 0)
    pltpu.sync_copy(r0.at[pl.ds(0, 32)], acc_sh.at[pl.ds(rbase + 1536, 32)])
    plsc.subcore_barrier()

    row0 = s * T * SUP

    trash = HALF + s

    def masks(ld, j):
        for k in range(CHUNK // 16):
            d = dstv[j, pl.ds(k * 16, 16)]
            mine = (d >= base) & (d < base + HALF)
            ld[pl.ds(k * 16, 16)] = jnp.where(mine, d - base, trash)

    def drain(buf, sem):
        pltpu.make_async_copy(h_hbm.at[pl.ds(0, CHUNK)], buf, sem).wait()

    def fire_scatter(k):
        pltpu.async_copy(R[k], acc_sh.at[LD[k]], SS[k], add=True)

    def step(t, carry):
        @pl.when(t > 0)
        def _():
            drain(r3, gs3)
            fire_scatter(3)
        r = row0 + t * SUP
        pltpu.sync_copy(src_hbm.at[pl.ds(r, SUP)], srcv)
        pltpu.sync_copy(dst_hbm.at[pl.ds(r, SUP)], dstv)
        for i in range(SUP):
            k = i % NSLOT
            if i >= NSLOT:
                drain(R[k], SS[k])
            else:
                @pl.when(t > 0)
                def _():
                    drain(R[k], SS[k])
            masks(LD[k], i)
            pltpu.async_copy(h_hbm.at[srcv.at[i]], R[k], GS[k])
            if i >= 1:
                kp = (i - 1) % NSLOT
                drain(R[kp], GS[kp])
                fire_scatter(kp)
        return carry
    lax.fori_loop(0, T, step, 0)
    drain(r3, gs3)
    fire_scatter(3)
    for k in range(NSLOT):
        drain(R[k], SS[k])
    plsc.subcore_barrier()

    pltpu.sync_copy(acc_sh.at[pl.ds(s * WB, WB)],
                    out_hbm.at[pl.ds(base + s * WB, WB)])

    @pl.when(s == 0)
    def _tail():
        pltpu.sync_copy(acc_sh.at[pl.ds(NS * WB, HALF - NS * WB)],
                        out_hbm.at[pl.ds(base + NS * WB, HALF - NS * WB)])


@jax.jit
def _sc_agg(h, gsrc, sdst):
    kfn = pl.kernel(
        _sc_agg_body,
        out_type=jax.ShapeDtypeStruct((N, HIDDEN), jnp.float32),
        mesh=plsc.VectorSubcoreMesh(core_axis_name="c", subcore_axis_name="s"),
        compiler_params=pltpu.CompilerParams(use_tc_tiling_on_sc=False),
        scratch_types=(
            [pltpu.VMEM((SUP, CHUNK), jnp.int32)] * 2
            + [pltpu.VMEM((CHUNK,), jnp.int32)] * 4
            + [pltpu.VMEM((CHUNK, HIDDEN), jnp.float32)] * 4
            + [pltpu.VMEM_SHARED((ACC_ROWS, HIDDEN), jnp.float32)]
            + [pltpu.SemaphoreType.DMA] * 8
        ),
    )
    return kfn(h, gsrc, sdst)


BLK = 2000


def _ln(x, g, b):
    mu = jnp.mean(x, axis=-1, keepdims=True)
    xc = x - mu
    var = jnp.mean(xc * xc, axis=-1, keepdims=True)
    return g * xc * lax.rsqrt(var + 1e-5) + b


def _gin_mlp_body(h_ref, agg_ref, w1_ref, b1_ref, g_ref, bt_ref,
                  w2_ref, b2_ref, dir_ref, lng_ref, lnb_ref, out_ref):
    h = h_ref[...]
    t = h + agg_ref[...]
    u = jnp.dot(t, w1_ref[...], preferred_element_type=jnp.float32) + b1_ref[...]
    u = jnp.maximum(u, 0.0)
    u = _ln(u, g_ref[...], bt_ref[...])
    v = jnp.dot(u, w2_ref[...], preferred_element_type=jnp.float32) + b2_ref[...]
    w = jnp.maximum(v + h + dir_ref[...], 0.0)
    out_ref[...] = _ln(w, lng_ref[...], lnb_ref[...])


def _row_spec(d):
    return pl.BlockSpec((BLK, d), lambda i: (i, 0))


def _full_spec(shape):
    nd = len(shape)
    return pl.BlockSpec(shape, lambda i: (0,) * nd)


@jax.jit
def _tc_gin_mlp(h, agg, w1, b1, g, bt, w2, b2, dirv, lng, lnb):
    return pl.pallas_call(
        _gin_mlp_body,
        grid=(N // BLK,),
        in_specs=[
            _row_spec(HIDDEN), _row_spec(HIDDEN),
            _full_spec((HIDDEN, HIDDEN)), _full_spec((1, HIDDEN)),
            _full_spec((1, HIDDEN)), _full_spec((1, HIDDEN)),
            _full_spec((HIDDEN, HIDDEN)), _full_spec((1, HIDDEN)),
            _full_spec((1, HIDDEN)), _full_spec((1, HIDDEN)),
            _full_spec((1, HIDDEN)),
        ],
        out_specs=_row_spec(HIDDEN),
        out_shape=jax.ShapeDtypeStruct((N, HIDDEN), jnp.float32),
    )(h, agg, w1, b1, g, bt, w2, b2, dirv, lng, lnb)


def _encode_body(x_ref, flag_ref, w_ref, wflag_ref, b_ref, out_ref):
    out_ref[...] = (
        jnp.dot(x_ref[...], w_ref[...], preferred_element_type=jnp.float32)
        + flag_ref[...] * wflag_ref[...] + b_ref[...])


@jax.jit
def _tc_encode(x_m, flag, w_x, w_flag, bias):
    return pl.pallas_call(
        _encode_body,
        grid=(N // BLK,),
        in_specs=[
            _row_spec(IN_DIM), _row_spec(1),
            _full_spec((IN_DIM, HIDDEN)), _full_spec((1, HIDDEN)),
            _full_spec((1, HIDDEN)),
        ],
        out_specs=_row_spec(HIDDEN),
        out_shape=jax.ShapeDtypeStruct((N, HIDDEN), jnp.float32),
    )(x_m, flag, w_x, w_flag, bias)


def _out_body(h_ref, w1_ref, b1_ref, g_ref, bt_ref, w2_ref, b2_ref, out_ref):
    u = jnp.dot(h_ref[...], w1_ref[...], preferred_element_type=jnp.float32)
    u = jnp.maximum(u + b1_ref[...], 0.0)
    u = _ln(u, g_ref[...], bt_ref[...])
    out_ref[...] = (
        jnp.dot(u, w2_ref[...], preferred_element_type=jnp.float32)
        + b2_ref[...])


@jax.jit
def _tc_out(h, w1, b1, g, bt, w2, b2):
    return pl.pallas_call(
        _out_body,
        grid=(N // BLK,),
        in_specs=[
            _row_spec(HIDDEN),
            _full_spec((HIDDEN, HIDDEN)), _full_spec((1, HIDDEN)),
            _full_spec((1, HIDDEN)), _full_spec((1, HIDDEN)),
            _full_spec((HIDDEN, IN_DIM)), _full_spec((1, IN_DIM)),
        ],
        out_specs=_row_spec(IN_DIM),
        out_shape=jax.ShapeDtypeStruct((N, IN_DIM), jnp.float32),
    )(h, w1, b1, g, bt, w2, b2)


def _r(v):
    return v.reshape(1, -1)


def _down_up(h, gsrc_f, sdst_f, gsrc_r, sdst_r, lp):
    p = lp["down"]
    agg = _sc_agg(h, gsrc_f, sdst_f)
    h = _tc_gin_mlp(h, agg, p["W1"], _r(p["b1"]), _r(p["g"]), _r(p["bt"]),
                    p["W2"], _r(p["b2"]), _r(lp["dir"][0]),
                    _r(lp["ln1g"]), _r(lp["ln1b"]))
    p = lp["up"]
    agg = _sc_agg(h, gsrc_r, sdst_r)
    h = _tc_gin_mlp(h, agg, p["W1"], _r(p["b1"]), _r(p["g"]), _r(p["bt"]),
                    p["W2"], _r(p["b2"]), _r(lp["dir"][1]),
                    _r(lp["ln2g"]), _r(lp["ln2b"]))
    return h


def kernel(x, edge_index, root_index, mask_idx, params):
    e = edge_index.shape[1]
    e_pad = NS * T * SUPE
    pad = e_pad - e
    src = edge_index[0]
    dst = edge_index[1]
    zpad = jnp.zeros((pad,), jnp.int32)
    npad = jnp.full((pad,), -1, jnp.int32)
    gsrc_f = jnp.concatenate([src, zpad]).reshape(-1, CHUNK)
    sdst_f = jnp.concatenate([dst, npad]).reshape(-1, CHUNK)
    gsrc_r = jnp.concatenate([dst, zpad]).reshape(-1, CHUNK)
    sdst_r = jnp.concatenate([src, npad]).reshape(-1, CHUNK)

    flag = jnp.zeros((N, 1), x.dtype).at[mask_idx].set(1.0)
    x_m = x.at[mask_idx].set(0.0)

    p = params
    h = _tc_encode(x_m, flag, p["enc_in_W"][:IN_DIM],
                   _r(p["enc_in_W"][IN_DIM]), _r(p["enc_in_b"]))
    for lp in p["enc_layers"]:
        h = _down_up(h, gsrc_f, sdst_f, gsrc_r, sdst_r, lp)

    z = jnp.take(h, root_index, axis=0)
    dec_bias = z @ p["dec_in_W"][IN_DIM + 1:] + _r(p["dec_in_b"])
    hd = _tc_encode(x_m, flag, p["dec_in_W"][:IN_DIM],
                    _r(p["dec_in_W"][IN_DIM]), dec_bias)
    for lp in p["dec_layers"]:
        hd = _down_up(hd, gsrc_f, sdst_f, gsrc_r, sdst_r, lp)

    return _tc_out(hd, p["out_W1"], _r(p["out_b1"]), _r(p["out_g"]),
                   _r(p["out_bt"]), p["out_W2"], _r(p["out_b2"]))


# speedup vs baseline: 1.2868x; 1.2868x over previous
import functools

import jax
import jax.numpy as jnp
from jax import lax
from jax.experimental import pallas as pl
from jax.experimental.pallas import tpu as pltpu
from jax.experimental.pallas import tpu_sc as plsc

N = 50000
IN_DIM = 19
HIDDEN = 64

NC = 2
NS = 16
HALF = 25000
ACC_ROWS = 25088
TRASH = 25080
CHUNK = 128
SUP = 8
SUPE = SUP * CHUNK
T = 49
WB = 1560
ROWS_PT = ACC_ROWS // NS


def _sc_agg_body(h_hbm, src_hbm, dst_hbm, out_hbm,
                 srcv, dstv, ld_a, ld_b, r_a, r_b,
                 acc_sh, gsa, gsb, ssa, ssb):
    c = lax.axis_index("c")
    s = lax.axis_index("s")
    base = c * HALF

    def zrow(i, carry):
        for j in range(HIDDEN // 16):
            r_a[i, pl.ds(j * 16, 16)] = jnp.zeros((16,), jnp.float32)
        return carry
    lax.fori_loop(0, CHUNK, zrow, 0)
    rbase = s * ROWS_PT

    def zsplat(i, carry):
        pltpu.sync_copy(r_a, acc_sh.at[pl.ds(rbase + i * CHUNK, CHUNK)])
        return carry
    lax.fori_loop(0, ROWS_PT // CHUNK, zsplat, 0)
    if ROWS_PT % CHUNK:
        pltpu.sync_copy(
            r_a.at[pl.ds(0, ROWS_PT % CHUNK)],
            acc_sh.at[pl.ds(rbase + ROWS_PT - ROWS_PT % CHUNK,
                            ROWS_PT % CHUNK)])
    plsc.subcore_barrier()

    row0 = s * T * SUP

    trash = HALF + s

    def masks(ld, j):
        for k in range(CHUNK // 16):
            d = dstv[j, pl.ds(k * 16, 16)]
            mine = (d >= base) & (d < base + HALF)
            ld[pl.ds(k * 16, 16)] = jnp.where(mine, d - base, trash)

    def drain(buf, sem):
        pltpu.make_async_copy(h_hbm.at[pl.ds(0, CHUNK)], buf, sem).wait()

    def step(t, carry):
        r = row0 + t * SUP
        pltpu.sync_copy(src_hbm.at[pl.ds(r, SUP)], srcv)
        pltpu.sync_copy(dst_hbm.at[pl.ds(r, SUP)], dstv)
        for i in range(0, SUP, 2):
            masks(ld_a, i)
            pltpu.async_copy(h_hbm.at[srcv.at[i]], r_a, gsa)
            masks(ld_b, i + 1)
            pltpu.async_copy(h_hbm.at[srcv.at[i + 1]], r_b, gsb)
            drain(r_a, gsa)
            pltpu.async_copy(r_a, acc_sh.at[ld_a], ssa, add=True)
            drain(r_a, ssa)
            drain(r_b, gsb)
            pltpu.async_copy(r_b, acc_sh.at[ld_b], ssb, add=True)
            drain(r_b, ssb)
        return carry
    lax.fori_loop(0, T, step, 0)
    plsc.subcore_barrier()

    pltpu.sync_copy(acc_sh.at[pl.ds(s * WB, WB)],
                    out_hbm.at[pl.ds(base + s * WB, WB)])

    @pl.when(s == 0)
    def _tail():
        pltpu.sync_copy(acc_sh.at[pl.ds(NS * WB, HALF - NS * WB)],
                        out_hbm.at[pl.ds(base + NS * WB, HALF - NS * WB)])


@jax.jit
def _sc_agg(h, gsrc, sdst):
    kfn = pl.kernel(
        _sc_agg_body,
        out_type=jax.ShapeDtypeStruct((N, HIDDEN), jnp.float32),
        mesh=plsc.VectorSubcoreMesh(core_axis_name="c", subcore_axis_name="s"),
        compiler_params=pltpu.CompilerParams(use_tc_tiling_on_sc=False),
        scratch_types=(
            [pltpu.VMEM((SUP, CHUNK), jnp.int32)] * 2
            + [pltpu.VMEM((CHUNK,), jnp.int32)] * 2
            + [pltpu.VMEM((CHUNK, HIDDEN), jnp.float32)] * 2
            + [pltpu.VMEM_SHARED((ACC_ROWS, HIDDEN), jnp.float32)]
            + [pltpu.SemaphoreType.DMA] * 4
        ),
    )
    return kfn(h, gsrc, sdst)


BLK = 2000


def _ln(x, g, b):
    mu = jnp.mean(x, axis=-1, keepdims=True)
    xc = x - mu
    var = jnp.mean(xc * xc, axis=-1, keepdims=True)
    return g * xc / jnp.sqrt(var + 1e-5) + b


def _gin_mlp_body(h_ref, agg_ref, w1_ref, b1_ref, g_ref, bt_ref,
                  w2_ref, b2_ref, dir_ref, lng_ref, lnb_ref, out_ref):
    h = h_ref[...]
    t = h + agg_ref[...]
    u = jnp.dot(t, w1_ref[...], preferred_element_type=jnp.float32) + b1_ref[...]
    u = jnp.maximum(u, 0.0)
    u = _ln(u, g_ref[...], bt_ref[...])
    v = jnp.dot(u, w2_ref[...], preferred_element_type=jnp.float32) + b2_ref[...]
    w = jnp.maximum(v + h + dir_ref[...], 0.0)
    out_ref[...] = _ln(w, lng_ref[...], lnb_ref[...])


def _row_spec(d):
    return pl.BlockSpec((BLK, d), lambda i: (i, 0))


def _full_spec(shape):
    nd = len(shape)
    return pl.BlockSpec(shape, lambda i: (0,) * nd)


@jax.jit
def _tc_gin_mlp(h, agg, w1, b1, g, bt, w2, b2, dirv, lng, lnb):
    return pl.pallas_call(
        _gin_mlp_body,
        grid=(N // BLK,),
        in_specs=[
            _row_spec(HIDDEN), _row_spec(HIDDEN),
            _full_spec((HIDDEN, HIDDEN)), _full_spec((1, HIDDEN)),
            _full_spec((1, HIDDEN)), _full_spec((1, HIDDEN)),
            _full_spec((HIDDEN, HIDDEN)), _full_spec((1, HIDDEN)),
            _full_spec((1, HIDDEN)), _full_spec((1, HIDDEN)),
            _full_spec((1, HIDDEN)),
        ],
        out_specs=_row_spec(HIDDEN),
        out_shape=jax.ShapeDtypeStruct((N, HIDDEN), jnp.float32),
    )(h, agg, w1, b1, g, bt, w2, b2, dirv, lng, lnb)


def _encode_body(x_ref, w_ref, b_ref, out_ref):
    out_ref[...] = (
        jnp.dot(x_ref[...], w_ref[...], preferred_element_type=jnp.float32)
        + b_ref[...])


@jax.jit
def _tc_encode(xcat, w, bias):
    d = xcat.shape[1]
    return pl.pallas_call(
        _encode_body,
        grid=(N // BLK,),
        in_specs=[
            _row_spec(d),
            _full_spec((d, HIDDEN)),
            _full_spec((1, HIDDEN)),
        ],
        out_specs=_row_spec(HIDDEN),
        out_shape=jax.ShapeDtypeStruct((N, HIDDEN), jnp.float32),
    )(xcat, w, bias)


def _out_body(h_ref, w1_ref, b1_ref, g_ref, bt_ref, w2_ref, b2_ref, out_ref):
    u = jnp.dot(h_ref[...], w1_ref[...], preferred_element_type=jnp.float32)
    u = jnp.maximum(u + b1_ref[...], 0.0)
    u = _ln(u, g_ref[...], bt_ref[...])
    out_ref[...] = (
        jnp.dot(u, w2_ref[...], preferred_element_type=jnp.float32)
        + b2_ref[...])


@jax.jit
def _tc_out(h, w1, b1, g, bt, w2, b2):
    return pl.pallas_call(
        _out_body,
        grid=(N // BLK,),
        in_specs=[
            _row_spec(HIDDEN),
            _full_spec((HIDDEN, HIDDEN)), _full_spec((1, HIDDEN)),
            _full_spec((1, HIDDEN)), _full_spec((1, HIDDEN)),
            _full_spec((HIDDEN, IN_DIM)), _full_spec((1, IN_DIM)),
        ],
        out_specs=_row_spec(IN_DIM),
        out_shape=jax.ShapeDtypeStruct((N, IN_DIM), jnp.float32),
    )(h, w1, b1, g, bt, w2, b2)


def _r(v):
    return v.reshape(1, -1)


def _down_up(h, gsrc_f, sdst_f, gsrc_r, sdst_r, lp):
    p = lp["down"]
    agg = _sc_agg(h, gsrc_f, sdst_f)
    h = _tc_gin_mlp(h, agg, p["W1"], _r(p["b1"]), _r(p["g"]), _r(p["bt"]),
                    p["W2"], _r(p["b2"]), _r(lp["dir"][0]),
                    _r(lp["ln1g"]), _r(lp["ln1b"]))
    p = lp["up"]
    agg = _sc_agg(h, gsrc_r, sdst_r)
    h = _tc_gin_mlp(h, agg, p["W1"], _r(p["b1"]), _r(p["g"]), _r(p["bt"]),
                    p["W2"], _r(p["b2"]), _r(lp["dir"][1]),
                    _r(lp["ln2g"]), _r(lp["ln2b"]))
    return h


def kernel(x, edge_index, root_index, mask_idx, params):
    e = edge_index.shape[1]
    e_pad = NS * T * SUPE
    pad = e_pad - e
    src = edge_index[0]
    dst = edge_index[1]
    zpad = jnp.zeros((pad,), jnp.int32)
    npad = jnp.full((pad,), -1, jnp.int32)
    gsrc_f = jnp.concatenate([src, zpad]).reshape(-1, CHUNK)
    sdst_f = jnp.concatenate([dst, npad]).reshape(-1, CHUNK)
    gsrc_r = jnp.concatenate([dst, zpad]).reshape(-1, CHUNK)
    sdst_r = jnp.concatenate([src, npad]).reshape(-1, CHUNK)

    flag = jnp.zeros((N, 1), x.dtype).at[mask_idx].set(1.0)
    x_m = x.at[mask_idx].set(0.0)

    p = params
    h = _tc_encode(jnp.concatenate([x_m, flag], axis=1),
                   p["enc_in_W"], _r(p["enc_in_b"]))
    for lp in p["enc_layers"]:
        h = _down_up(h, gsrc_f, sdst_f, gsrc_r, sdst_r, lp)

    z = jnp.take(h, root_index, axis=0)
    z_node = jnp.broadcast_to(z, (N, HIDDEN))
    hd = _tc_encode(jnp.concatenate([x_m, flag, z_node], axis=1),
                    p["dec_in_W"], _r(p["dec_in_b"]))
    for lp in p["dec_layers"]:
        hd = _down_up(hd, gsrc_f, sdst_f, gsrc_r, sdst_r, lp)

    return _tc_out(hd, p["out_W1"], _r(p["out_b1"]), _r(p["out_g"]),
                   _r(p["out_bt"]), p["out_W2"], _r(p["out_b2"]))
